# Initial kernel scaffold; baseline (speedup 1.0000x reference)
import jax
import jax.numpy as jnp
from jax.experimental import pallas as pl


def kernel(user_ids, item_ids, items_count):
    raise NotImplementedError("write your pallas kernel here")


# speedup vs baseline: 1.0449x; 1.0449x over previous
import functools

import jax
import jax.numpy as jnp
from jax import lax
from jax.experimental import pallas as pl
from jax.experimental.pallas import tpu as pltpu
from jax.experimental.pallas import tpu_sc as plsc

_CH = 128


@functools.lru_cache(maxsize=None)
def _make_sc_gather(B):
    info = plsc.get_sparse_core_info()
    NW = info.num_cores * info.num_subcores
    NC = info.num_cores
    assert B % (NW * _CH) == 0
    n_ch = B // (NW * _CH)
    rows = B // _CH

    mesh = plsc.VectorSubcoreMesh(core_axis_name="c", subcore_axis_name="s")

    @functools.partial(
        pl.kernel,
        mesh=mesh,
        out_type=jax.ShapeDtypeStruct((rows, _CH), jnp.float32),
        scratch_types=[
            pltpu.VMEM((n_ch, _CH), jnp.int32),
            pltpu.VMEM((n_ch, _CH), jnp.float32),
            pltpu.SemaphoreType.DMA,
        ],
    )
    def gather_k(table_hbm, idx_hbm, out_hbm, idx_v, vals_v, sem):
        wid = lax.axis_index("s") * NC + lax.axis_index("c")
        base = wid * n_ch
        pltpu.sync_copy(idx_hbm.at[pl.ds(base, n_ch)], idx_v)
        copies = []
        for j in range(n_ch):
            copies.append(
                pltpu.async_copy(table_hbm.at[idx_v.at[j]], vals_v.at[j], sem)
            )
        for c in copies:
            c.wait()
        pltpu.sync_copy(vals_v, out_hbm.at[pl.ds(base, n_ch)])

    return gather_k


def _bcast_body(r_ref, o_ref):
    o_ref[...] = jnp.broadcast_to(r_ref[...], o_ref.shape)


@functools.lru_cache(maxsize=None)
def _make_bcast(n_users, B):
    row_blk = 128
    grid = n_users // row_blk
    return pl.pallas_call(
        _bcast_body,
        grid=(grid,),
        in_specs=[pl.BlockSpec((1, B), lambda i: (0, 0))],
        out_specs=pl.BlockSpec((row_blk, B), lambda i: (i, 0)),
        out_shape=jax.ShapeDtypeStruct((n_users, B), jnp.float32),
    )


def kernel(user_ids, item_ids, items_count):
    n_users = user_ids.shape[0]
    B = item_ids.shape[0]
    idx2d = item_ids.reshape(-1, _CH)
    ratings = _make_sc_gather(B)(items_count, idx2d)
    return _make_bcast(n_users, B)(ratings.reshape(1, B))


# speedup vs baseline: 1.0454x; 1.0004x over previous
import functools

import jax
import jax.numpy as jnp
from jax import lax
from jax.experimental import pallas as pl
from jax.experimental.pallas import tpu as pltpu
from jax.experimental.pallas import tpu_sc as plsc

_CH = 128


@functools.lru_cache(maxsize=None)
def _make_sc_gather(B):
    info = plsc.get_sparse_core_info()
    NW = info.num_cores * info.num_subcores
    NC = info.num_cores
    assert B % (NW * _CH) == 0
    n_ch = B // (NW * _CH)
    rows = B // _CH

    mesh = plsc.VectorSubcoreMesh(core_axis_name="c", subcore_axis_name="s")

    @functools.partial(
        pl.kernel,
        mesh=mesh,
        out_type=jax.ShapeDtypeStruct((rows, _CH), jnp.float32),
        scratch_types=[
            pltpu.VMEM((n_ch, _CH), jnp.int32),
            pltpu.VMEM((n_ch, _CH), jnp.float32),
            pltpu.SemaphoreType.DMA,
        ],
    )
    def gather_k(table_hbm, idx_hbm, out_hbm, idx_v, vals_v, sem):
        wid = lax.axis_index("s") * NC + lax.axis_index("c")
        base = wid * n_ch
        pltpu.sync_copy(idx_hbm.at[pl.ds(base, n_ch)], idx_v)
        copies = []
        for j in range(n_ch):
            copies.append(
                pltpu.async_copy(table_hbm.at[idx_v.at[j]], vals_v.at[j], sem)
            )
        for c in copies:
            c.wait()
        pltpu.sync_copy(vals_v, out_hbm.at[pl.ds(base, n_ch)])

    return gather_k


_RB = 32


@functools.lru_cache(maxsize=None)
def _make_bcast(n_users, B):
    n_dma = n_users // _RB

    def _bcast_body(r_ref, o_ref, buf, sem):
        buf[...] = jnp.broadcast_to(r_ref[...], buf.shape)
        copies = [
            pltpu.make_async_copy(buf, o_ref.at[pl.ds(i * _RB, _RB), :], sem)
            for i in range(n_dma)
        ]
        for c in copies:
            c.start()
        for c in copies:
            c.wait()

    return pl.pallas_call(
        _bcast_body,
        in_specs=[pl.BlockSpec(memory_space=pltpu.VMEM)],
        out_specs=pl.BlockSpec(memory_space=pl.ANY),
        out_shape=jax.ShapeDtypeStruct((n_users, B), jnp.float32),
        scratch_shapes=[
            pltpu.VMEM((_RB, B), jnp.float32),
            pltpu.SemaphoreType.DMA,
        ],
    )


def kernel(user_ids, item_ids, items_count):
    n_users = user_ids.shape[0]
    B = item_ids.shape[0]
    idx2d = item_ids.reshape(-1, _CH)
    ratings = _make_sc_gather(B)(items_count, idx2d)
    return _make_bcast(n_users, B)(ratings.reshape(1, B))
